# Initial kernel scaffold; baseline (speedup 1.0000x reference)
import jax
import jax.numpy as jnp
from jax.experimental import pallas as pl


def kernel(x, laplacian, bn1_gamma, bn1_beta, W1, b1, bn2_gamma, bn2_beta, W2, b2):
    raise NotImplementedError("write your pallas kernel here")


# speedup vs baseline: 1.0724x; 1.0724x over previous
import jax
import jax.numpy as jnp
from jax.experimental import pallas as pl
from jax.experimental.pallas import tpu as pltpu

_N = 4096
_C = 256
_EPS = 1e-5
_B = 64
_G = _N // _B
_T = 512
_NT = _N // _T


def _bn_coeffs(v, gamma, beta):
    mean = jnp.mean(v, axis=0, keepdims=True)
    var = jnp.mean((v - mean) ** 2, axis=0, keepdims=True)
    rstd = gamma / jnp.sqrt(var + _EPS)
    return rstd, beta - mean * rstd


def _body(lap_ref, x_ref, w1_ref, b1_ref, g1_ref, be1_ref,
          w2_ref, b2_ref, g2_ref, be2_ref,
          out_ref, lbf, xhbf, t1bf, t2bf, hf, accf):
    g = pl.program_id(0)

    @pl.when(g == 0)
    def _():
        s1, o1 = _bn_coeffs(x_ref[...], g1_ref[...], be1_ref[...])
        xhbf[...] = (x_ref[...] * s1 + o1).astype(jnp.bfloat16)

    blk = lap_ref[...].astype(jnp.bfloat16)
    lbf[pl.ds(g * _B, _B), :] = blk
    t1bf[pl.ds(g * _B, _B), :] = jnp.dot(
        blk, xhbf[...], preferred_element_type=jnp.float32
    ).astype(jnp.bfloat16)

    @pl.when(g == _G - 1)
    def _():
        w1 = w1_ref[...]
        w2 = w2_ref[...]
        s1, o1 = _bn_coeffs(x_ref[...], g1_ref[...], be1_ref[...])

        def mm(a, b):
            return jnp.dot(a, b, preferred_element_type=jnp.float32)

        def rows(ref, i):
            return ref[pl.ds(i * _T, _T), :]

        for i in range(_NT):
            xh_t = rows(x_ref, i) * s1 + o1
            t2_t = 2.0 * mm(rows(lbf, i), t1bf[...]) - xh_t
            t2c = t2_t.astype(jnp.bfloat16)
            t2bf[pl.ds(i * _T, _T), :] = t2c
            accf[pl.ds(i * _T, _T), :] = (
                mm(rows(xhbf, i), w1[0]) + mm(rows(t1bf, i), w1[1]) + mm(t2c, w1[2])
            )
        for i in range(_NT):
            t3_t = 2.0 * mm(rows(lbf, i), t2bf[...]) - rows(t1bf, i).astype(jnp.float32)
            hf[pl.ds(i * _T, _T), :] = jnp.maximum(
                rows(accf, i) + mm(t3_t.astype(jnp.bfloat16), w1[3]) + b1_ref[...],
                0.0,
            )
        s2, o2 = _bn_coeffs(hf[...], g2_ref[...], be2_ref[...])
        for i in range(_NT):
            hb_c = (rows(hf, i) * s2 + o2).astype(jnp.bfloat16)
            t1bf[pl.ds(i * _T, _T), :] = hb_c
            accf[pl.ds(i * _T, _T), :] = mm(hb_c, w2[0])
        for i in range(_NT):
            u1_t = mm(rows(lbf, i), t1bf[...])
            u1c = u1_t.astype(jnp.bfloat16)
            t2bf[pl.ds(i * _T, _T), :] = u1c
            accf[pl.ds(i * _T, _T), :] = rows(accf, i) + mm(u1c, w2[1])
        for i in range(_NT):
            hb_t = rows(hf, i) * s2 + o2
            u2_t = 2.0 * mm(rows(lbf, i), t2bf[...]) - hb_t
            u2c = u2_t.astype(jnp.bfloat16)
            xhbf[pl.ds(i * _T, _T), :] = u2c
            accf[pl.ds(i * _T, _T), :] = rows(accf, i) + mm(u2c, w2[2])
        for i in range(_NT):
            u3_t = 2.0 * mm(rows(lbf, i), xhbf[...]) - rows(t2bf, i).astype(jnp.float32)
            xh_t = rows(x_ref, i) * s1 + o1
            out_ref[pl.ds(i * _T, _T), :] = jnp.maximum(
                xh_t + rows(accf, i) + mm(u3_t.astype(jnp.bfloat16), w2[3]) + b2_ref[...],
                0.0,
            )


def kernel(x, laplacian, bn1_gamma, bn1_beta, W1, b1, bn2_gamma, bn2_beta, W2, b2):
    w1b = W1.astype(jnp.bfloat16)
    w2b = W2.astype(jnp.bfloat16)
    b1r = b1.reshape(1, _C)
    b2r = b2.reshape(1, _C)
    g1r = bn1_gamma.reshape(1, _C)
    be1r = bn1_beta.reshape(1, _C)
    g2r = bn2_gamma.reshape(1, _C)
    be2r = bn2_beta.reshape(1, _C)

    full = lambda shape: pl.BlockSpec(shape, lambda g: tuple(0 for _ in shape))
    return pl.pallas_call(
        _body,
        grid=(_G,),
        in_specs=[
            pl.BlockSpec((_B, _N), lambda g: (g, 0)),
            full((_N, _C)),
            full((4, _C, _C)),
            full((1, _C)),
            full((1, _C)),
            full((1, _C)),
            full((4, _C, _C)),
            full((1, _C)),
            full((1, _C)),
            full((1, _C)),
        ],
        out_specs=full((_N, _C)),
        out_shape=jax.ShapeDtypeStruct((_N, _C), jnp.float32),
        scratch_shapes=[
            pltpu.VMEM((_N, _N), jnp.bfloat16),
            pltpu.VMEM((_N, _C), jnp.bfloat16),
            pltpu.VMEM((_N, _C), jnp.bfloat16),
            pltpu.VMEM((_N, _C), jnp.bfloat16),
            pltpu.VMEM((_N, _C), jnp.float32),
            pltpu.VMEM((_N, _C), jnp.float32),
        ],
        compiler_params=pltpu.CompilerParams(
            dimension_semantics=("arbitrary",),
            vmem_limit_bytes=64 * 1024 * 1024,
        ),
    )(laplacian, x, w1b, b1r, g1r, be1r, w2b, b2r, g2r, be2r)
